# baseline (speedup 1.0000x reference)
import dataclasses
import functools

import jax
import jax.numpy as jnp
from jax import lax
from jax.experimental import pallas as pl
from jax.experimental.pallas import tpu as pltpu
from jax.experimental.pallas import tpu_sc as plsc

_N, _E, _D = 10000, 320000, 128
_BLK = 1000

_NC, _NS = 2, 16
_NW = _NC * _NS
_EW = _E // _NW
_SUB = 125
_JN = 8
_K = _JN * _SUB
_NCH = _EW // _K
_IRW = _EW // _SUB
_NPAD = 10240
_RPS = _NPAD // _NS
_RPS_LAST = _N - (_NS - 1) * _RPS


def _copy_out(src_sp, dst_hbm, c, s):
    out_off = c * _N + s * _RPS

    @pl.when(s < _NS - 1)
    def _():
        pltpu.sync_copy(src_sp.at[pl.ds(s * _RPS, _RPS)],
                        dst_hbm.at[pl.ds(out_off, _RPS)])

    @pl.when(s == _NS - 1)
    def _():
        pltpu.sync_copy(src_sp.at[pl.ds(s * _RPS, _RPS_LAST)],
                        dst_hbm.at[pl.ds(out_off, _RPS_LAST)])


def _sc_agg_body(x_hbm, src_hbm, dst_hbm, z_hbm, agg_out,
                 sidx_v, didx_v, rows_v, acc_sp, gs0, gs1, ss0, ss1):
    gs = (gs0, gs1)
    ss = (ss0, ss1)
    c = lax.axis_index("c")
    s = lax.axis_index("s")
    wid = c * _NS + s

    pltpu.sync_copy(z_hbm, acc_sp.at[pl.ds(s * _RPS, _RPS)])
    plsc.subcore_barrier()

    row_base = wid * _IRW

    def _scatter_drain(b, j):
        pltpu.make_async_copy(
            rows_v.at[b], acc_sp.at[didx_v.at[j]], ss[b]).wait()

    @pl.loop(0, _NCH)
    def _(t):
        @pl.when(t > 0)
        def _():
            _scatter_drain(0, _JN - 2)
            _scatter_drain(1, _JN - 1)

        r0 = row_base + t * _JN
        pltpu.sync_copy(src_hbm.at[pl.ds(r0, _JN)], sidx_v)
        pltpu.sync_copy(dst_hbm.at[pl.ds(r0, _JN)], didx_v)
        cps = [None] * _JN
        for j in range(_JN):
            b = j % 2
            if j >= 2:
                _scatter_drain(b, j - 2)
            cps[j] = pltpu.async_copy(x_hbm.at[sidx_v.at[j]], rows_v.at[b],
                                      gs[b])
            if j >= 1:
                cps[j - 1].wait()
                pltpu.async_copy(rows_v.at[(j - 1) % 2],
                                 acc_sp.at[didx_v.at[j - 1]], ss[(j - 1) % 2],
                                 add=True)
        cps[_JN - 1].wait()
        pltpu.async_copy(rows_v.at[(_JN - 1) % 2],
                         acc_sp.at[didx_v.at[_JN - 1]], ss[(_JN - 1) % 2],
                         add=True)

    _scatter_drain(0, _JN - 2)
    _scatter_drain(1, _JN - 1)
    plsc.subcore_barrier()
    _copy_out(acc_sp, agg_out, c, s)


_DROWS = _E // _D
_DCHUNKS = _DROWS // _JN
_DTAIL = _DROWS - _DCHUNKS * _JN


def _sc_deg_body(dst_hbm, deg_out, didx_v, acc_v):
    c = lax.axis_index("c")
    s = lax.axis_index("s")
    wid = c * _NS + s

    @pl.loop(0, _N // 16)
    def _(i):
        acc_v[pl.ds(i * 16, 16)] = jnp.zeros((16,), jnp.float32)

    ones16 = jnp.ones((16,), jnp.float32)

    @pl.loop(0, (_DCHUNKS + _NW - 1) // _NW)
    def _(t):
        chunk = wid + t * _NW

        @pl.when(chunk < _DCHUNKS)
        def _():
            pltpu.sync_copy(dst_hbm.at[pl.ds(chunk * _JN, _JN)], didx_v)
            for r in range(_JN):
                for q in range(_D // 16):
                    idx16 = didx_v[r, pl.ds(q * 16, 16)]
                    plsc.addupdate_scatter(acc_v, [idx16], ones16)

    @pl.when(wid == _NW - 1)
    def _():
        pltpu.sync_copy(dst_hbm.at[pl.ds(_DCHUNKS * _JN, _DTAIL)],
                        didx_v.at[pl.ds(0, _DTAIL)])
        for r in range(_DTAIL):
            for q in range(_D // 16):
                idx16 = didx_v[r, pl.ds(q * 16, 16)]
                plsc.addupdate_scatter(acc_v, [idx16], ones16)

    for b in range(_N // _BLK):
        pltpu.sync_copy(acc_v.at[pl.ds(b * _BLK, _BLK)],
                        deg_out.at[pl.ds((b * _NW + wid) * _BLK, _BLK)])


_sc_mesh = plsc.VectorSubcoreMesh(core_axis_name="c", subcore_axis_name="s")

_sc_agg = pl.kernel(
    _sc_agg_body,
    out_type=[jax.ShapeDtypeStruct((2 * _N, _D), jnp.float32)],
    mesh=_sc_mesh,
    scratch_types=[
        pltpu.VMEM((_JN, _SUB), jnp.int32),
        pltpu.VMEM((_JN, _SUB), jnp.int32),
        pltpu.VMEM((2, _SUB, _D), jnp.float32),
        pltpu.VMEM_SHARED((_NPAD, _D), jnp.float32),
        pltpu.SemaphoreType.DMA,
        pltpu.SemaphoreType.DMA,
        pltpu.SemaphoreType.DMA,
        pltpu.SemaphoreType.DMA,
    ],
)

_sc_deg = pl.kernel(
    _sc_deg_body,
    out_type=[jax.ShapeDtypeStruct((_NW * _N,), jnp.float32)],
    mesh=_sc_mesh,
    scratch_types=[
        pltpu.VMEM((_JN, _D), jnp.int32),
        pltpu.VMEM((_N,), jnp.float32),
    ],
    compiler_params=dataclasses.replace(
        pltpu.CompilerParams(), needs_layout_passes=False),
)


def _dense_body(relu, aggp_ref, degp_ref, x_ref, Wl_ref, b_ref, Wr_ref, out_ref):
    agg = aggp_ref[0] + aggp_ref[1]
    deg = jnp.sum(degp_ref[0], axis=0)[:, None]
    recip = 1.0 / jnp.maximum(deg, 1.0)
    mean = agg * recip
    y = (jnp.dot(mean, Wl_ref[...], preferred_element_type=jnp.float32)
         + b_ref[...]
         + jnp.dot(x_ref[...], Wr_ref[...], preferred_element_type=jnp.float32))
    out_ref[...] = jnp.maximum(y, 0.0) if relu else y


def _dense_layer(aggp, degp, x, Wl, b, Wr, relu):
    grid = (_N // _BLK,)
    return pl.pallas_call(
        functools.partial(_dense_body, relu),
        grid=grid,
        in_specs=[
            pl.BlockSpec((2, _BLK, _D), lambda i: (0, i, 0)),
            pl.BlockSpec((1, _NW, _BLK), lambda i: (i, 0, 0)),
            pl.BlockSpec((_BLK, _D), lambda i: (i, 0)),
            pl.BlockSpec((_D, _D), lambda i: (0, 0)),
            pl.BlockSpec((1, _D), lambda i: (0, 0)),
            pl.BlockSpec((_D, _D), lambda i: (0, 0)),
        ],
        out_specs=pl.BlockSpec((_BLK, _D), lambda i: (i, 0)),
        out_shape=jax.ShapeDtypeStruct((_N, _D), jnp.float32),
    )(aggp, degp, x, Wl, b.reshape(1, _D), Wr)


def kernel(x, edge_index, W1l, b1, W1r, W2l, b2, W2r):
    src2d = edge_index[0].reshape(_E // _SUB, _SUB)
    dst2d = edge_index[1].reshape(_E // _SUB, _SUB)
    dst128 = edge_index[1].reshape(_DROWS, _D)
    zrows = jnp.zeros((_RPS, _D), jnp.float32)

    degp = _sc_deg(dst128)[0].reshape(_N // _BLK, _NW, _BLK)
    agg1 = _sc_agg(x, src2d, dst2d, zrows)[0].reshape(2, _N, _D)
    h = _dense_layer(agg1, degp, x, W1l, b1, W1r, relu=True)
    agg2 = _sc_agg(h, src2d, dst2d, zrows)[0].reshape(2, _N, _D)
    out = _dense_layer(agg2, degp, h, W2l, b2, W2r, relu=False)
    return out


# speedup vs baseline: 1.0048x; 1.0048x over previous
import dataclasses
import functools

import jax
import jax.numpy as jnp
from jax import lax
from jax.experimental import pallas as pl
from jax.experimental.pallas import tpu as pltpu
from jax.experimental.pallas import tpu_sc as plsc

_N, _E, _D = 10000, 320000, 128
_BLK = 1000

_NC, _NS = 2, 16
_NW = _NC * _NS
_EW = _E // _NW
_SUB = 125
_JN = 8
_K = _JN * _SUB
_NCH = _EW // _K
_IRW = _EW // _SUB
_NPAD = 10240
_RPS = _NPAD // _NS
_RPS_LAST = _N - (_NS - 1) * _RPS


def _copy_out(src_sp, dst_hbm, c, s):
    out_off = c * _N + s * _RPS

    @pl.when(s < _NS - 1)
    def _():
        pltpu.sync_copy(src_sp.at[pl.ds(s * _RPS, _RPS)],
                        dst_hbm.at[pl.ds(out_off, _RPS)])

    @pl.when(s == _NS - 1)
    def _():
        pltpu.sync_copy(src_sp.at[pl.ds(s * _RPS, _RPS_LAST)],
                        dst_hbm.at[pl.ds(out_off, _RPS_LAST)])


_DROWS = _E // _D
_DCHUNKS = _DROWS // _JN
_DTAIL = _DROWS - _DCHUNKS * _JN


def _hist_rows(didx128_v, hist_v, nrows):
    ones16 = jnp.ones((16,), jnp.float32)
    for r in range(nrows):
        for q in range(_D // 16):
            idx16 = didx128_v[r, pl.ds(q * 16, 16)]
            plsc.addupdate_scatter(hist_v, [idx16], ones16)


def _sc_agg_body(with_deg, *refs):
    if with_deg:
        (x_hbm, src_hbm, dst_hbm, dst128_hbm, z_hbm, agg_out, deg_out,
         sidx_v, didx_v, rows_v, didx128_v, hist_v, acc_sp,
         gs0, gs1, ss0, ss1) = refs
    else:
        (x_hbm, src_hbm, dst_hbm, z_hbm, agg_out,
         sidx_v, didx_v, rows_v, acc_sp, gs0, gs1, ss0, ss1) = refs
    gs = (gs0, gs1)
    ss = (ss0, ss1)
    c = lax.axis_index("c")
    s = lax.axis_index("s")
    wid = c * _NS + s

    pltpu.sync_copy(z_hbm, acc_sp.at[pl.ds(s * _RPS, _RPS)])
    if with_deg:
        @pl.loop(0, _N // 16)
        def _(i):
            hist_v[pl.ds(i * 16, 16)] = jnp.zeros((16,), jnp.float32)
    plsc.subcore_barrier()

    row_base = wid * _IRW

    def _scatter_drain(b, j):
        pltpu.make_async_copy(
            rows_v.at[b], acc_sp.at[didx_v.at[j]], ss[b]).wait()

    @pl.loop(0, _NCH)
    def _(t):
        @pl.when(t > 0)
        def _():
            _scatter_drain(0, _JN - 2)
            _scatter_drain(1, _JN - 1)

        r0 = row_base + t * _JN
        pltpu.sync_copy(src_hbm.at[pl.ds(r0, _JN)], sidx_v)
        pltpu.sync_copy(dst_hbm.at[pl.ds(r0, _JN)], didx_v)
        cps = [None] * _JN
        for j in range(_JN):
            b = j % 2
            if j >= 2:
                _scatter_drain(b, j - 2)
            cps[j] = pltpu.async_copy(x_hbm.at[sidx_v.at[j]], rows_v.at[b],
                                      gs[b])
            if with_deg and j == 1:
                chunk = wid + t * _NW

                @pl.when(chunk < _DCHUNKS)
                def _():
                    pltpu.sync_copy(
                        dst128_hbm.at[pl.ds(chunk * _JN, _JN)], didx128_v)
                    _hist_rows(didx128_v, hist_v, _JN)
            if j >= 1:
                cps[j - 1].wait()
                pltpu.async_copy(rows_v.at[(j - 1) % 2],
                                 acc_sp.at[didx_v.at[j - 1]], ss[(j - 1) % 2],
                                 add=True)
        cps[_JN - 1].wait()
        pltpu.async_copy(rows_v.at[(_JN - 1) % 2],
                         acc_sp.at[didx_v.at[_JN - 1]], ss[(_JN - 1) % 2],
                         add=True)

    _scatter_drain(0, _JN - 2)
    _scatter_drain(1, _JN - 1)
    if with_deg:
        @pl.when(wid == _NW - 1)
        def _():
            pltpu.sync_copy(dst128_hbm.at[pl.ds(_DCHUNKS * _JN, _DTAIL)],
                            didx128_v.at[pl.ds(0, _DTAIL)])
            _hist_rows(didx128_v, hist_v, _DTAIL)
        for b in range(_N // _BLK):
            pltpu.sync_copy(hist_v.at[pl.ds(b * _BLK, _BLK)],
                            deg_out.at[pl.ds((b * _NW + wid) * _BLK, _BLK)])
    plsc.subcore_barrier()
    _copy_out(acc_sp, agg_out, c, s)


_sc_mesh = plsc.VectorSubcoreMesh(core_axis_name="c", subcore_axis_name="s")


def _make_sc_agg(with_deg):
    out_type = [jax.ShapeDtypeStruct((2 * _N, _D), jnp.float32)]
    scratch = [
        pltpu.VMEM((_JN, _SUB), jnp.int32),
        pltpu.VMEM((_JN, _SUB), jnp.int32),
        pltpu.VMEM((2, _SUB, _D), jnp.float32),
    ]
    if with_deg:
        out_type.append(jax.ShapeDtypeStruct((_NW * _N,), jnp.float32))
        scratch.append(pltpu.VMEM((_JN, _D), jnp.int32))
        scratch.append(pltpu.VMEM((_N,), jnp.float32))
    scratch.append(pltpu.VMEM_SHARED((_NPAD, _D), jnp.float32))
    scratch.extend([pltpu.SemaphoreType.DMA] * 4)
    cp = pltpu.CompilerParams()
    if with_deg:
        cp = dataclasses.replace(cp, needs_layout_passes=False)
    return pl.kernel(
        functools.partial(_sc_agg_body, with_deg),
        out_type=out_type,
        mesh=_sc_mesh,
        scratch_types=scratch,
        compiler_params=cp,
    )


_sc_agg_deg = _make_sc_agg(True)
_sc_agg = _make_sc_agg(False)


def _dense_body(relu, aggp_ref, degp_ref, x_ref, Wl_ref, b_ref, Wr_ref, out_ref):
    agg = aggp_ref[0] + aggp_ref[1]
    deg = jnp.sum(degp_ref[0], axis=0)[:, None]
    recip = 1.0 / jnp.maximum(deg, 1.0)
    mean = agg * recip
    y = (jnp.dot(mean, Wl_ref[...], preferred_element_type=jnp.float32)
         + b_ref[...]
         + jnp.dot(x_ref[...], Wr_ref[...], preferred_element_type=jnp.float32))
    out_ref[...] = jnp.maximum(y, 0.0) if relu else y


def _dense_layer(aggp, degp, x, Wl, b, Wr, relu):
    grid = (_N // _BLK,)
    return pl.pallas_call(
        functools.partial(_dense_body, relu),
        grid=grid,
        in_specs=[
            pl.BlockSpec((2, _BLK, _D), lambda i: (0, i, 0)),
            pl.BlockSpec((1, _NW, _BLK), lambda i: (i, 0, 0)),
            pl.BlockSpec((_BLK, _D), lambda i: (i, 0)),
            pl.BlockSpec((_D, _D), lambda i: (0, 0)),
            pl.BlockSpec((1, _D), lambda i: (0, 0)),
            pl.BlockSpec((_D, _D), lambda i: (0, 0)),
        ],
        out_specs=pl.BlockSpec((_BLK, _D), lambda i: (i, 0)),
        out_shape=jax.ShapeDtypeStruct((_N, _D), jnp.float32),
    )(aggp, degp, x, Wl, b.reshape(1, _D), Wr)


def kernel(x, edge_index, W1l, b1, W1r, W2l, b2, W2r):
    src2d = edge_index[0].reshape(_E // _SUB, _SUB)
    dst2d = edge_index[1].reshape(_E // _SUB, _SUB)
    dst128 = edge_index[1].reshape(_DROWS, _D)
    zrows = jnp.zeros((_RPS, _D), jnp.float32)

    agg1, degp = _sc_agg_deg(x, src2d, dst2d, dst128, zrows)
    agg1 = agg1.reshape(2, _N, _D)
    degp = degp.reshape(_N // _BLK, _NW, _BLK)
    h = _dense_layer(agg1, degp, x, W1l, b1, W1r, relu=True)
    agg2 = _sc_agg(h, src2d, dst2d, zrows)[0].reshape(2, _N, _D)
    out = _dense_layer(agg2, degp, h, W2l, b2, W2r, relu=False)
    return out


# speedup vs baseline: 1.0728x; 1.0677x over previous
import dataclasses
import functools

import jax
import jax.numpy as jnp
from jax import lax
from jax.experimental import pallas as pl
from jax.experimental.pallas import tpu as pltpu
from jax.experimental.pallas import tpu_sc as plsc

_N, _E, _D = 10000, 320000, 128
_BLK = 1000

_NC, _NS = 2, 16
_NW = _NC * _NS
_EW = _E // _NW
_SUB = 125
_JN = 16
_K = _JN * _SUB
_NCH = _EW // _K
_IRW = _EW // _SUB
_RPS = 624
_RPS_LAST = _N - (_NS - 1) * _RPS


def _acc_slice(s):
    return s * _RPS


def _copy_out(src_sp, dst_hbm, c, s):
    out_off = c * _N + s * _RPS

    @pl.when(s < _NS - 1)
    def _():
        pltpu.sync_copy(src_sp.at[pl.ds(s * _RPS, _RPS)],
                        dst_hbm.at[pl.ds(out_off, _RPS)])

    @pl.when(s == _NS - 1)
    def _():
        pltpu.sync_copy(src_sp.at[pl.ds(s * _RPS, _RPS_LAST)],
                        dst_hbm.at[pl.ds(out_off, _RPS_LAST)])


_DJN = 8
_DROWS = _E // _D
_DCHUNKS = _DROWS // _DJN
_DTAIL = _DROWS - _DCHUNKS * _DJN


def _hist_rows(didx128_v, hist_v, nrows):
    ones16 = jnp.ones((16,), jnp.float32)
    for r in range(nrows):
        for q in range(_D // 16):
            idx16 = didx128_v[r, pl.ds(q * 16, 16)]
            plsc.addupdate_scatter(hist_v, [idx16], ones16)


def _sc_agg_body(with_deg, *refs):
    if with_deg:
        (x_hbm, src_hbm, dst_hbm, dst128_hbm, z_hbm, agg_out, deg_out,
         sidx_v, didx_v, rows_v, didx128_v, hist_v, acc_sp,
         gs0, gs1, ss0, ss1) = refs
    else:
        (x_hbm, src_hbm, dst_hbm, z_hbm, agg_out,
         sidx_v, didx_v, rows_v, acc_sp, gs0, gs1, ss0, ss1) = refs
    gs = (gs0, gs1)
    ss = (ss0, ss1)
    c = lax.axis_index("c")
    s = lax.axis_index("s")
    wid = c * _NS + s

    @pl.when(s < _NS - 1)
    def _():
        pltpu.sync_copy(z_hbm.at[pl.ds(0, _RPS)],
                        acc_sp.at[pl.ds(s * _RPS, _RPS)])

    @pl.when(s == _NS - 1)
    def _():
        pltpu.sync_copy(z_hbm, acc_sp.at[pl.ds(s * _RPS, _RPS_LAST)])

    if with_deg:
        @pl.loop(0, _N // 16)
        def _(i):
            hist_v[pl.ds(i * 16, 16)] = jnp.zeros((16,), jnp.float32)
    plsc.subcore_barrier()

    row_base = wid * _IRW

    def _scatter_drain(b, j):
        pltpu.make_async_copy(
            rows_v.at[b], acc_sp.at[didx_v.at[j]], ss[b]).wait()

    @pl.loop(0, _NCH)
    def _(t):
        @pl.when(t > 0)
        def _():
            _scatter_drain(0, _JN - 2)
            _scatter_drain(1, _JN - 1)

        r0 = row_base + t * _JN
        pltpu.sync_copy(src_hbm.at[pl.ds(r0, _JN)], sidx_v)
        pltpu.sync_copy(dst_hbm.at[pl.ds(r0, _JN)], didx_v)
        cps = [None] * _JN
        for j in range(_JN):
            b = j % 2
            if j >= 2:
                _scatter_drain(b, j - 2)
            cps[j] = pltpu.async_copy(x_hbm.at[sidx_v.at[j]], rows_v.at[b],
                                      gs[b])
            if with_deg and j in (1, 8):
                chunk = wid + (2 * t + (j == 8)) * _NW

                @pl.when(chunk < _DCHUNKS)
                def _():
                    pltpu.sync_copy(
                        dst128_hbm.at[pl.ds(chunk * _DJN, _DJN)], didx128_v)
                    _hist_rows(didx128_v, hist_v, _DJN)
            if j >= 1:
                cps[j - 1].wait()
                pltpu.async_copy(rows_v.at[(j - 1) % 2],
                                 acc_sp.at[didx_v.at[j - 1]], ss[(j - 1) % 2],
                                 add=True)
        cps[_JN - 1].wait()
        pltpu.async_copy(rows_v.at[(_JN - 1) % 2],
                         acc_sp.at[didx_v.at[_JN - 1]], ss[(_JN - 1) % 2],
                         add=True)

    _scatter_drain(0, _JN - 2)
    _scatter_drain(1, _JN - 1)
    if with_deg:
        @pl.when(wid == _NW - 1)
        def _():
            pltpu.sync_copy(dst128_hbm.at[pl.ds(_DCHUNKS * _DJN, _DTAIL)],
                            didx128_v.at[pl.ds(0, _DTAIL)])
            _hist_rows(didx128_v, hist_v, _DTAIL)
        for b in range(_N // _BLK):
            pltpu.sync_copy(hist_v.at[pl.ds(b * _BLK, _BLK)],
                            deg_out.at[pl.ds((b * _NW + wid) * _BLK, _BLK)])
    plsc.subcore_barrier()
    _copy_out(acc_sp, agg_out, c, s)


_sc_mesh = plsc.VectorSubcoreMesh(core_axis_name="c", subcore_axis_name="s")


def _make_sc_agg(with_deg):
    out_type = [jax.ShapeDtypeStruct((2 * _N, _D), jnp.float32)]
    scratch = [
        pltpu.VMEM((_JN, _SUB), jnp.int32),
        pltpu.VMEM((_JN, _SUB), jnp.int32),
        pltpu.VMEM((2, _SUB, _D), jnp.float32),
    ]
    if with_deg:
        out_type.append(jax.ShapeDtypeStruct((_NW * _N,), jnp.float32))
        scratch.append(pltpu.VMEM((_DJN, _D), jnp.int32))
        scratch.append(pltpu.VMEM((_N,), jnp.float32))
    scratch.append(pltpu.VMEM_SHARED((_N, _D), jnp.float32))
    scratch.extend([pltpu.SemaphoreType.DMA] * 4)
    cp = pltpu.CompilerParams()
    if with_deg:
        cp = dataclasses.replace(cp, needs_layout_passes=False)
    return pl.kernel(
        functools.partial(_sc_agg_body, with_deg),
        out_type=out_type,
        mesh=_sc_mesh,
        scratch_types=scratch,
        compiler_params=cp,
    )


_sc_agg_deg = _make_sc_agg(True)
_sc_agg = _make_sc_agg(False)


def _dense_body(relu, aggp_ref, degp_ref, x_ref, Wl_ref, b_ref, Wr_ref, out_ref):
    agg = aggp_ref[0] + aggp_ref[1]
    deg = jnp.sum(degp_ref[0], axis=0)[:, None]
    recip = 1.0 / jnp.maximum(deg, 1.0)
    mean = agg * recip
    y = (jnp.dot(mean, Wl_ref[...], preferred_element_type=jnp.float32)
         + b_ref[...]
         + jnp.dot(x_ref[...], Wr_ref[...], preferred_element_type=jnp.float32))
    out_ref[...] = jnp.maximum(y, 0.0) if relu else y


def _dense_layer(aggp, degp, x, Wl, b, Wr, relu):
    grid = (_N // _BLK,)
    return pl.pallas_call(
        functools.partial(_dense_body, relu),
        grid=grid,
        in_specs=[
            pl.BlockSpec((2, _BLK, _D), lambda i: (0, i, 0)),
            pl.BlockSpec((1, _NW, _BLK), lambda i: (i, 0, 0)),
            pl.BlockSpec((_BLK, _D), lambda i: (i, 0)),
            pl.BlockSpec((_D, _D), lambda i: (0, 0)),
            pl.BlockSpec((1, _D), lambda i: (0, 0)),
            pl.BlockSpec((_D, _D), lambda i: (0, 0)),
        ],
        out_specs=pl.BlockSpec((_BLK, _D), lambda i: (i, 0)),
        out_shape=jax.ShapeDtypeStruct((_N, _D), jnp.float32),
    )(aggp, degp, x, Wl, b.reshape(1, _D), Wr)


def kernel(x, edge_index, W1l, b1, W1r, W2l, b2, W2r):
    src2d = edge_index[0].reshape(_E // _SUB, _SUB)
    dst2d = edge_index[1].reshape(_E // _SUB, _SUB)
    dst128 = edge_index[1].reshape(_DROWS, _D)
    zrows = jnp.zeros((_RPS_LAST, _D), jnp.float32)

    agg1, degp = _sc_agg_deg(x, src2d, dst2d, dst128, zrows)
    agg1 = agg1.reshape(2, _N, _D)
    degp = degp.reshape(_N // _BLK, _NW, _BLK)
    h = _dense_layer(agg1, degp, x, W1l, b1, W1r, relu=True)
    agg2 = _sc_agg(h, src2d, dst2d, zrows)[0].reshape(2, _N, _D)
    out = _dense_layer(agg2, degp, h, W2l, b2, W2r, relu=False)
    return out
